# Initial kernel scaffold; baseline (speedup 1.0000x reference)
import jax
import jax.numpy as jnp
from jax.experimental import pallas as pl


def kernel(x, edge_index, W1, b1, gamma1, beta1, W2, b2, gamma2, beta2):
    raise NotImplementedError("write your pallas kernel here")


# speedup vs baseline: 8.4978x; 8.4978x over previous
import functools

import jax
import jax.numpy as jnp
from jax import lax
from jax.experimental import pallas as pl
from jax.experimental.pallas import tpu as pltpu
from jax.experimental.pallas import tpu_sc as plsc

NS = 16
NC = 2
C = 128


def _ceil_to(a, m):
    return -(-a // m) * m


def _make_deg_kernel(n, kch_half, d_rows, r_d):
    mesh = plsc.VectorSubcoreMesh(core_axis_name="c", subcore_axis_name="s")
    kch = kch_half * NC

    @functools.partial(
        pl.kernel,
        out_type=[jax.ShapeDtypeStruct((d_rows,), jnp.float32),
                  jax.ShapeDtypeStruct((d_rows,), jnp.float32)],
        mesh=mesh,
        scratch_types=[
            pltpu.VMEM_SHARED((d_rows,), jnp.float32),
            pltpu.VMEM((kch, C), jnp.int32),
            pltpu.VMEM((C,), jnp.float32),
            pltpu.VMEM((r_d,), jnp.float32),
        ],
    )
    def deg_kernel(dst3_hbm, ones_hbm, zeros_hbm, d0_hbm, d1_hbm,
                   dacc, dstv, onesv, zbuf):
        c = lax.axis_index("c")
        s = lax.axis_index("s")
        pltpu.sync_copy(zeros_hbm, zbuf)
        pltpu.sync_copy(zbuf, dacc.at[pl.ds(s * r_d, r_d)])
        pltpu.sync_copy(ones_hbm.at[pl.ds(0, C)], onesv)
        pltpu.sync_copy(dst3_hbm.at[s], dstv)
        plsc.subcore_barrier()

        def body(kk, carry):
            k = c * kch_half + kk
            pltpu.sync_copy(onesv, dacc.at[dstv.at[k]], add=True)
            return carry

        lax.fori_loop(0, kch_half, body, 0)
        plsc.subcore_barrier()
        pltpu.sync_copy(dacc.at[pl.ds(s * r_d, r_d)], zbuf)

        @pl.when(c == 0)
        def _():
            pltpu.sync_copy(zbuf, d0_hbm.at[pl.ds(s * r_d, r_d)])

        @pl.when(c == 1)
        def _():
            pltpu.sync_copy(zbuf, d1_hbm.at[pl.ds(s * r_d, r_d)])

    return deg_kernel


def _make_agg_kernel(n, h, kch, acc_rows, r_z, r_w):
    mesh = plsc.VectorSubcoreMesh(core_axis_name="c", subcore_axis_name="s")

    @functools.partial(
        pl.kernel,
        out_type=jax.ShapeDtypeStruct((2 * n, h), jnp.float32),
        mesh=mesh,
        scratch_types=[
            pltpu.VMEM_SHARED((acc_rows, h), jnp.float32),
            pltpu.VMEM((kch, C), jnp.int32),
            pltpu.VMEM((2, C), jnp.int32),
            pltpu.VMEM((C, h), jnp.float32),
            pltpu.VMEM((C, h), jnp.float32),
            pltpu.SemaphoreType.DMA,
            pltpu.SemaphoreType.DMA,
            pltpu.SemaphoreType.DMA,
            pltpu.SemaphoreType.DMA,
        ],
    )
    def agg_kernel(h_hbm, src3a_hbm, src3b_hbm, dst3_hbm, zeros_hbm, out_hbm,
                   acc, srcv, di, rows_a, rows_b, sem_a, sem_b, sem_d0,
                   sem_d1):
        c = lax.axis_index("c")
        s = lax.axis_index("s")

        @pl.when(c == 0)
        def _():
            pltpu.sync_copy(src3a_hbm.at[s], srcv)

        @pl.when(c == 1)
        def _():
            pltpu.sync_copy(src3b_hbm.at[s], srcv)

        pltpu.sync_copy(zeros_hbm, rows_a)
        off = 0
        while off < r_z:
            cnt = min(C, r_z - off)
            pltpu.sync_copy(rows_a.at[pl.ds(0, cnt)],
                            acc.at[pl.ds(s * r_z + off, cnt)])
            off += cnt

        pltpu.async_copy(dst3_hbm.at[s, 0], di.at[0], sem_d0)
        pltpu.async_copy(dst3_hbm.at[s, 1], di.at[1], sem_d1)
        pltpu.async_copy(h_hbm.at[srcv.at[0]], rows_a, sem_a)
        pltpu.async_copy(h_hbm.at[srcv.at[1]], rows_b, sem_b)
        plsc.subcore_barrier()

        def stage(k, rows, sem, di_slot, sem_d):
            pltpu.make_async_copy(h_hbm.at[srcv.at[k]], rows, sem).wait()
            pltpu.make_async_copy(dst3_hbm.at[s, k], di_slot, sem_d).wait()
            pltpu.sync_copy(rows, acc.at[di_slot], add=True)

            @pl.when(k + 2 < kch)
            def _():
                pltpu.async_copy(dst3_hbm.at[s, k + 2], di_slot, sem_d)
                pltpu.async_copy(h_hbm.at[srcv.at[k + 2]], rows, sem)

        def body(t, carry):
            stage(2 * t, rows_a, sem_a, di.at[0], sem_d0)
            stage(2 * t + 1, rows_b, sem_b, di.at[1], sem_d1)
            return carry

        lax.fori_loop(0, kch // 2, body, 0)
        plsc.subcore_barrier()

        def writeout(base, nrows):
            off = 0
            while off < nrows:
                cnt = min(C, nrows - off)
                pltpu.sync_copy(acc.at[pl.ds(base + off, cnt)],
                                rows_a.at[pl.ds(0, cnt)])
                pltpu.sync_copy(rows_a.at[pl.ds(0, cnt)],
                                out_hbm.at[pl.ds(c * n + base + off, cnt)])
                off += cnt

        @pl.when(s < NS - 1)
        def _():
            writeout(s * r_w, r_w)

        @pl.when(s == NS - 1)
        def _():
            writeout((NS - 1) * r_w, n - (NS - 1) * r_w)

    return agg_kernel


def _k1_body(n, x_ref, w_ref, d0_ref, d1_ref, hp_ref, dinv_ref):
    deg = d0_ref[...] + d1_ref[...] + 1.0
    dinv = lax.rsqrt(deg)
    hp = jnp.dot(x_ref[...], w_ref[...], preferred_element_type=jnp.float32)
    hp_ref[...] = hp * dinv
    dinv_ref[...] = dinv


def _k_stats_body(agg_ref, hp_ref, dinv_ref, pre_ref, s_ref, q_ref):
    i = pl.program_id(1)
    pre = (agg_ref[...] + hp_ref[...]) * dinv_ref[...]
    pre_ref[...] = pre
    ps = jnp.sum(pre, axis=0)[None, None, :]
    pq = jnp.sum(pre * pre, axis=0)[None, None, :]

    @pl.when(i == 0)
    def _():
        s_ref[...] = ps
        q_ref[...] = pq

    @pl.when(i > 0)
    def _():
        s_ref[...] += ps
        q_ref[...] += pq


def _bn_scale_shift(n, s_ref, q_ref, g_ref, b_ref, eps=1e-5):
    mean = s_ref[...] / n
    var = q_ref[...] / n - mean * mean
    scale = lax.rsqrt(var + eps) * g_ref[...]
    shift = b_ref[...] - mean * scale
    return scale, shift


def _k_mid_body(n, h, p0_ref, p1_ref, s_ref, q_ref, g_ref, b_ref, w_ref,
                dinv_ref, hp2_ref):
    scale, shift = _bn_scale_shift(n, s_ref, q_ref, g_ref, b_ref)
    z0 = jnp.maximum(p0_ref[...] * scale[0] + shift[0], 0.0)
    z1 = jnp.maximum(p1_ref[...] * scale[1] + shift[1], 0.0)
    h2 = (jnp.dot(z0, w_ref[:h, :], preferred_element_type=jnp.float32)
          + jnp.dot(z1, w_ref[h:, :], preferred_element_type=jnp.float32))
    hp2_ref[...] = h2 * dinv_ref[...]


def _k_final_body(n, p0_ref, p1_ref, s_ref, q_ref, g_ref, b_ref, out_ref):
    scale, shift = _bn_scale_shift(n, s_ref, q_ref, g_ref, b_ref)
    z0 = p0_ref[...] * scale[0] + shift[0]
    z1 = p1_ref[...] * scale[1] + shift[1]
    out_ref[...] = jnp.concatenate([z0, z1], axis=1)


def kernel(x, edge_index, W1, b1, gamma1, beta1, W2, b2, gamma2, beta2):
    n, d = x.shape
    e = edge_index.shape[1]
    h = d // 2

    kch = _ceil_to(-(-e // (NS * C)), 2)
    e_pad = NS * kch * C
    src = jnp.concatenate(
        [edge_index[0], jnp.zeros((e_pad - e,), jnp.int32)])
    dst = jnp.concatenate(
        [edge_index[1], jnp.full((e_pad - e,), n, jnp.int32)])
    src3a = src.reshape(NS, kch, C)
    src3b = src3a + n
    dst3 = dst.reshape(NS, kch, C)

    r_z = _ceil_to(-(-(n + 1) // NS), 8)
    acc_rows = NS * r_z
    r_w = (n // NS) // 8 * 8
    r_d = _ceil_to(-(-(n + 1) // NS), 128)
    d_rows = NS * r_d

    zeros_rows = jnp.zeros((C, h), jnp.float32)
    zeros_vec = jnp.zeros((r_d,), jnp.float32)
    ones_vec = jnp.ones((r_d,), jnp.float32)

    deg_kernel = _make_deg_kernel(n, kch // NC, d_rows, r_d)
    d0, d1 = deg_kernel(dst3, ones_vec, zeros_vec)
    d0s = d0[:n].reshape(n, 1)
    d1s = d1[:n].reshape(n, 1)

    R = 2000
    nb = n // R
    hp1, dinv = pl.pallas_call(
        functools.partial(_k1_body, n),
        grid=(nb, 2),
        in_specs=[
            pl.BlockSpec((R, d), lambda i, j: (i, 0)),
            pl.BlockSpec((d, h), lambda i, j: (0, j)),
            pl.BlockSpec((R, 1), lambda i, j: (i, 0)),
            pl.BlockSpec((R, 1), lambda i, j: (i, 0)),
        ],
        out_specs=[
            pl.BlockSpec((R, h), lambda i, j: (j * nb + i, 0)),
            pl.BlockSpec((R, 1), lambda i, j: (i, 0)),
        ],
        out_shape=[
            jax.ShapeDtypeStruct((2 * n, h), jnp.float32),
            jax.ShapeDtypeStruct((n, 1), jnp.float32),
        ],
    )(x, W1, d0s, d1s)

    agg_kernel = _make_agg_kernel(n, h, kch, acc_rows, r_z, r_w)

    def stats_call(agg, hp):
        return pl.pallas_call(
            _k_stats_body,
            grid=(2, nb),
            in_specs=[
                pl.BlockSpec((R, h), lambda c, i: (c * nb + i, 0)),
                pl.BlockSpec((R, h), lambda c, i: (c * nb + i, 0)),
                pl.BlockSpec((R, 1), lambda c, i: (i, 0)),
            ],
            out_specs=[
                pl.BlockSpec((R, h), lambda c, i: (c * nb + i, 0)),
                pl.BlockSpec((1, 1, h), lambda c, i: (c, 0, 0)),
                pl.BlockSpec((1, 1, h), lambda c, i: (c, 0, 0)),
            ],
            out_shape=[
                jax.ShapeDtypeStruct((2 * n, h), jnp.float32),
                jax.ShapeDtypeStruct((2, 1, h), jnp.float32),
                jax.ShapeDtypeStruct((2, 1, h), jnp.float32),
            ],
        )(agg, hp, dinv)

    agg1 = agg_kernel(hp1, src3a, src3b, dst3, zeros_rows)
    pre1, s1, q1 = stats_call(agg1, hp1)

    g1 = gamma1.reshape(2, 1, h)
    be1 = beta1.reshape(2, 1, h)
    hp2 = pl.pallas_call(
        functools.partial(_k_mid_body, n, h),
        grid=(nb, 2),
        in_specs=[
            pl.BlockSpec((R, h), lambda i, j: (i, 0)),
            pl.BlockSpec((R, h), lambda i, j: (nb + i, 0)),
            pl.BlockSpec((2, 1, h), lambda i, j: (0, 0, 0)),
            pl.BlockSpec((2, 1, h), lambda i, j: (0, 0, 0)),
            pl.BlockSpec((2, 1, h), lambda i, j: (0, 0, 0)),
            pl.BlockSpec((2, 1, h), lambda i, j: (0, 0, 0)),
            pl.BlockSpec((d, h), lambda i, j: (0, j)),
            pl.BlockSpec((R, 1), lambda i, j: (i, 0)),
        ],
        out_specs=pl.BlockSpec((R, h), lambda i, j: (j * nb + i, 0)),
        out_shape=jax.ShapeDtypeStruct((2 * n, h), jnp.float32),
    )(pre1, pre1, s1, q1, g1, be1, W2, dinv)

    agg2 = agg_kernel(hp2, src3a, src3b, dst3, zeros_rows)
    pre2, s2, q2 = stats_call(agg2, hp2)

    g2 = gamma2.reshape(2, 1, h)
    be2 = beta2.reshape(2, 1, h)
    out = pl.pallas_call(
        functools.partial(_k_final_body, n),
        grid=(nb,),
        in_specs=[
            pl.BlockSpec((R, h), lambda i: (i, 0)),
            pl.BlockSpec((R, h), lambda i: (nb + i, 0)),
            pl.BlockSpec((2, 1, h), lambda i: (0, 0, 0)),
            pl.BlockSpec((2, 1, h), lambda i: (0, 0, 0)),
            pl.BlockSpec((2, 1, h), lambda i: (0, 0, 0)),
            pl.BlockSpec((2, 1, h), lambda i: (0, 0, 0)),
        ],
        out_specs=pl.BlockSpec((R, d), lambda i: (i, 0)),
        out_shape=jax.ShapeDtypeStruct((n, d), jnp.float32),
    )(pre2, pre2, s2, q2, g2, be2)
    return out
